# baseline (speedup 1.0000x reference)
import functools

import numpy as np
import jax
import jax.numpy as jnp
from jax import lax
from jax.experimental import pallas as pl
from jax.experimental.pallas import tpu as pltpu
from jax.experimental.pallas import tpu_sc as plsc

N = 262144
N_LEVELS = 16
LOG2_T = 19
T = 1 << LOG2_T
MASK = T - 1
SCALE = float(np.exp((np.log(4096.0) - np.log(16.0)) / (N_LEVELS - 1)))
RES = [float(np.floor(16.0 * SCALE ** l)) for l in range(N_LEVELS)]
P1I = int(np.uint32(2654435761).astype(np.int32))
P2I = int(np.uint32(805459861).astype(np.int32))

NW = 32
SPT = N // NW
B = 128
NG = B // 16
NCHUNK = SPT // B


def _sc_encode_body(table_hbm, posT_hbm, out_hbm, xn_v,
                    i0, i1, i2, i3, i4, i5, i6, i7,
                    rem_v, w_v, f01_v, enc_v, sem):
    idx_refs = (i0, i1, i2, i3, i4, i5, i6, i7)
    wid = lax.axis_index("s") * 2 + lax.axis_index("c")
    base0 = wid * SPT

    def chunk_body(ci, carry):
        cbase = base0 + ci * B
        pltpu.sync_copy(posT_hbm.at[:, pl.ds(cbase, B)], xn_v)

        def norm_body(g, c):
            for d in range(3):
                p = xn_v[d, pl.ds(g * 16, 16)]
                xn_v[d, pl.ds(g * 16, 16)] = (p + 1.0) * 0.5
            return c

        lax.fori_loop(0, NG, norm_body, 0)

        for l in range(N_LEVELS):
            res = RES[l]
            lbase = l * T

            def idx_body(g, c, res=res, lbase=lbase):
                pid = []
                fr = []
                for d in range(3):
                    posd = xn_v[d, pl.ds(g * 16, 16)] * res
                    pi = posd.astype(jnp.int32)
                    pid.append(pi)
                    fr.append(posd - pi.astype(jnp.float32))
                hx = pid[0]
                hy = pid[1] * P1I
                hz = pid[2] * P2I
                hx1 = hx + 1
                hy1 = hy + P1I
                hz1 = hz + P2I
                exy = [hx ^ hy, hx1 ^ hy, hx ^ hy1, hx1 ^ hy1]
                fx1, fy1, fz1 = fr
                fx0 = 1.0 - fx1
                fy0 = 1.0 - fy1
                fz0 = 1.0 - fz1
                wxy = [fx0 * fy0, fx1 * fy0, fx0 * fy1, fx1 * fy1]
                for cc in range(8):
                    oz = (cc >> 2) & 1
                    k = cc & 3
                    h = exy[k] ^ (hz1 if oz else hz)
                    e = (h & MASK) + lbase
                    idx_refs[cc][pl.ds(g * 16, 16)] = e >> 3
                    rem_v[pl.ds(cc * B + g * 16, 16)] = (e & 7) << 1
                    w_v[pl.ds(cc * B + g * 16, 16)] = wxy[k] * (fz1 if oz else fz0)
                return c

            lax.fori_loop(0, NG, idx_body, 0)

            for cc in range(8):
                pltpu.async_copy(
                    table_hbm.at[idx_refs[cc]],
                    f01_v.at[pl.ds(cc * B, B)],
                    sem,
                )
            pltpu.make_async_copy(
                table_hbm.at[pl.ds(0, 8 * B)], f01_v, sem
            ).wait()

            lanes = lax.iota(jnp.int32, 16)

            def comb_body(g, c, l=l):
                acc0 = jnp.zeros((16,), jnp.float32)
                acc1 = jnp.zeros((16,), jnp.float32)
                for cc in range(8):
                    rem2 = rem_v[pl.ds(cc * B + g * 16, 16)]
                    wv = w_v[pl.ds(cc * B + g * 16, 16)]
                    rowv = (cc * B + g * 16) + lanes
                    e0 = plsc.load_gather(f01_v, [rowv, rem2])
                    e1 = plsc.load_gather(f01_v, [rowv, rem2 + 1])
                    acc0 = acc0 + wv * e0
                    acc1 = acc1 + wv * e1
                enc_v[2 * l, pl.ds(g * 16, 16)] = acc0
                enc_v[2 * l + 1, pl.ds(g * 16, 16)] = acc1
                return c

            lax.fori_loop(0, NG, comb_body, 0)

        pltpu.sync_copy(enc_v, out_hbm.at[:, pl.ds(cbase, B)])
        return carry

    lax.fori_loop(0, NCHUNK, chunk_body, 0)


_SC_SCRATCH = [
    pltpu.VMEM((3, B), jnp.float32),
    *[pltpu.VMEM((B,), jnp.int32) for _ in range(8)],
    pltpu.VMEM((8 * B,), jnp.int32),
    pltpu.VMEM((8 * B,), jnp.float32),
    pltpu.VMEM((8 * B, 16), jnp.float32),
    pltpu.VMEM((2 * N_LEVELS, B), jnp.float32),
    pltpu.SemaphoreType.DMA,
]


def _make_sc_encode(interpret=False):
    return functools.partial(
        pl.kernel,
        out_type=jax.ShapeDtypeStruct((2 * N_LEVELS, N), jnp.float32),
        mesh=plsc.VectorSubcoreMesh(
            core_axis_name="c", subcore_axis_name="s", num_cores=2, num_subcores=16
        ),
        scratch_types=_SC_SCRATCH,
        compiler_params=pltpu.CompilerParams(
            needs_layout_passes=False, use_tc_tiling_on_sc=False
        ),
        interpret=interpret,
    )(_sc_encode_body)


_sc_encode = _make_sc_encode()


BN = 1024


def _mlp_body(encT, posT, dirT, w1t, w2t, hw1t, hw2t, hw3t, rgbT, denT):
    e = encT[...]
    h = jnp.maximum(jnp.dot(w1t[...], e, preferred_element_type=jnp.float32), 0.0)
    out = jnp.dot(w2t[...], h, preferred_element_type=jnp.float32)

    p = posT[...]
    x = (p + 1.0) / 2.0
    x0, x1, x2 = x[0:1, :], x[1:2, :], x[2:3, :]
    sel = (x0 > 0.0) & (x0 < 1.0) & (x1 > 0.0) & (x1 < 1.0) & (x2 > 0.0) & (x2 < 1.0)
    den = jnp.exp(out[0:1, :] - 1.0) * sel.astype(jnp.float32)

    d = dirT[...]
    dx, dy, dz = d[0:1, :], d[1:2, :], d[2:3, :]
    xx, yy, zz = dx * dx, dy * dy, dz * dz
    comps = [
        jnp.full_like(dx, 0.28209479177387814),
        -0.48860251190291987 * dy,
        0.48860251190291987 * dz,
        -0.48860251190291987 * dx,
        1.0925484305920792 * dx * dy,
        -1.0925484305920792 * dy * dz,
        0.94617469575755997 * zz - 0.31539156525251999,
        -1.0925484305920792 * dx * dz,
        0.54627421529603959 * (xx - yy),
        0.59004358992664352 * dy * (3.0 * xx - yy),
        2.8906114426405538 * dx * dy * dz,
        0.45704579946446572 * dy * (4.0 * zz - xx - yy),
        0.3731763325901154 * dz * (2.0 * zz - 3.0 * xx - 3.0 * yy),
        0.45704579946446572 * dx * (4.0 * zz - xx - yy),
        1.4453057213202769 * dz * (xx - yy),
        0.59004358992664352 * dx * (xx - 3.0 * yy),
    ]
    sh = jnp.concatenate(comps, axis=0)
    hh = jnp.concatenate([sh, out[1:16, :], jnp.zeros((1, BN), jnp.float32)], axis=0)
    h1 = jnp.maximum(jnp.dot(hw1t[...], hh, preferred_element_type=jnp.float32), 0.0)
    h2 = jnp.maximum(jnp.dot(hw2t[...], h1, preferred_element_type=jnp.float32), 0.0)
    z = jnp.dot(hw3t[...], h2, preferred_element_type=jnp.float32)
    rgbT[...] = 1.0 / (1.0 + jnp.exp(-z))
    denT[...] = den


def _mlp(encT, posT, dirT, w1t, w2t, hw1t, hw2t, hw3t):
    grid = (N // BN,)
    return pl.pallas_call(
        _mlp_body,
        grid=grid,
        in_specs=[
            pl.BlockSpec((2 * N_LEVELS, BN), lambda i: (0, i)),
            pl.BlockSpec((3, BN), lambda i: (0, i)),
            pl.BlockSpec((3, BN), lambda i: (0, i)),
            pl.BlockSpec((64, 32), lambda i: (0, 0)),
            pl.BlockSpec((16, 64), lambda i: (0, 0)),
            pl.BlockSpec((64, 32), lambda i: (0, 0)),
            pl.BlockSpec((64, 64), lambda i: (0, 0)),
            pl.BlockSpec((3, 64), lambda i: (0, 0)),
        ],
        out_specs=[
            pl.BlockSpec((3, BN), lambda i: (0, i)),
            pl.BlockSpec((1, BN), lambda i: (0, i)),
        ],
        out_shape=[
            jax.ShapeDtypeStruct((3, N), jnp.float32),
            jax.ShapeDtypeStruct((1, N), jnp.float32),
        ],
    )(encT, posT, dirT, w1t, w2t, hw1t, hw2t, hw3t)


def kernel(positions, directions, hash_table, base_W1, base_W2, head_W1, head_W2, head_W3):
    posT = positions.T
    dirT = directions.T
    table16 = hash_table.reshape(N_LEVELS * T * 2 // 16, 16)
    encT = _sc_encode(table16, posT)
    w1t = base_W1.T
    w2t = base_W2.T
    hw1t = jnp.pad(head_W1, ((0, 1), (0, 0))).T
    hw2t = head_W2.T
    hw3t = head_W3.T
    rgbT, denT = _mlp(encT, posT, dirT, w1t, w2t, hw1t, hw2t, hw3t)
    return rgbT.T, denT.T


# speedup vs baseline: 5.1777x; 5.1777x over previous
import functools

import numpy as np
import jax
import jax.numpy as jnp
from jax import lax
from jax.experimental import pallas as pl
from jax.experimental.pallas import tpu as pltpu
from jax.experimental.pallas import tpu_sc as plsc

N = 262144
N_LEVELS = 16
LOG2_T = 19
T = 1 << LOG2_T
MASK = T - 1
SCALE = float(np.exp((np.log(4096.0) - np.log(16.0)) / (N_LEVELS - 1)))
RES = [float(np.floor(16.0 * SCALE ** l)) for l in range(N_LEVELS)]
P1I = int(np.uint32(2654435761).astype(np.int32))
P2I = int(np.uint32(805459861).astype(np.int32))

NW = 32
SPT = N // NW
B = 128
NG = B // 16
NCHUNK = SPT // B


def _sc_encode_body(table_hbm, posT_hbm, out_hbm, xn_v,
                    i0, i1, i2, i3, i4, i5, i6, i7,
                    w_v, words_v, enc_v, sem):
    idx_refs = (i0, i1, i2, i3, i4, i5, i6, i7)
    wid = lax.axis_index("s") * 2 + lax.axis_index("c")
    base0 = wid * SPT

    def chunk_body(ci, carry):
        cbase = base0 + ci * B
        pltpu.sync_copy(posT_hbm.at[:, pl.ds(cbase, B)], xn_v)

        def norm_body(g, c):
            for d in range(3):
                p = xn_v[d, pl.ds(g * 16, 16)]
                xn_v[d, pl.ds(g * 16, 16)] = (p + 1.0) * 0.5
            return c

        lax.fori_loop(0, NG, norm_body, 0)

        for l in range(N_LEVELS):
            res = RES[l]
            lbase = l * T

            def idx_body(g, c, res=res, lbase=lbase):
                pid = []
                fr = []
                for d in range(3):
                    posd = xn_v[d, pl.ds(g * 16, 16)] * res
                    pi = posd.astype(jnp.int32)
                    pid.append(pi)
                    fr.append(posd - pi.astype(jnp.float32))
                hx = pid[0]
                hy = pid[1] * P1I
                hz = pid[2] * P2I
                hx1 = hx + 1
                hy1 = hy + P1I
                hz1 = hz + P2I
                exy = [hx ^ hy, hx1 ^ hy, hx ^ hy1, hx1 ^ hy1]
                fx1, fy1, fz1 = fr
                fx0 = 1.0 - fx1
                fy0 = 1.0 - fy1
                fz0 = 1.0 - fz1
                wxy = [fx0 * fy0, fx1 * fy0, fx0 * fy1, fx1 * fy1]
                for cc in range(8):
                    oz = (cc >> 2) & 1
                    k = cc & 3
                    h = exy[k] ^ (hz1 if oz else hz)
                    idx_refs[cc][pl.ds(g * 16, 16)] = (h & MASK) + lbase
                    w_v[pl.ds(cc * B + g * 16, 16)] = wxy[k] * (fz1 if oz else fz0)
                return c

            lax.fori_loop(0, NG, idx_body, 0)

            for cc in range(8):
                pltpu.async_copy(
                    table_hbm.at[idx_refs[cc]],
                    words_v.at[pl.ds(cc * B, B)],
                    sem,
                )
            pltpu.make_async_copy(
                table_hbm.at[pl.ds(0, 8 * B)], words_v, sem
            ).wait()

            def comb_body(g, c, l=l):
                acc0 = jnp.zeros((16,), jnp.float32)
                acc1 = jnp.zeros((16,), jnp.float32)
                for cc in range(8):
                    wv = w_v[pl.ds(cc * B + g * 16, 16)]
                    word = words_v[pl.ds(cc * B + g * 16, 16)]
                    bf = plsc.bitcast(word, jnp.bfloat16)
                    e0, e1 = plsc.unpack(bf, format=plsc.PackFormat.INTERLEAVED)
                    acc0 = acc0 + wv * e0
                    acc1 = acc1 + wv * e1
                enc_v[2 * l, pl.ds(g * 16, 16)] = acc0
                enc_v[2 * l + 1, pl.ds(g * 16, 16)] = acc1
                return c

            lax.fori_loop(0, NG, comb_body, 0)

        pltpu.sync_copy(enc_v, out_hbm.at[:, pl.ds(cbase, B)])
        return carry

    lax.fori_loop(0, NCHUNK, chunk_body, 0)


_SC_SCRATCH = [
    pltpu.VMEM((3, B), jnp.float32),
    *[pltpu.VMEM((B,), jnp.int32) for _ in range(8)],
    pltpu.VMEM((8 * B,), jnp.float32),
    pltpu.VMEM((8 * B,), jnp.int32),
    pltpu.VMEM((2 * N_LEVELS, B), jnp.float32),
    pltpu.SemaphoreType.DMA,
]


def _make_sc_encode(interpret=False):
    return functools.partial(
        pl.kernel,
        out_type=jax.ShapeDtypeStruct((2 * N_LEVELS, N), jnp.float32),
        mesh=plsc.VectorSubcoreMesh(
            core_axis_name="c", subcore_axis_name="s", num_cores=2, num_subcores=16
        ),
        scratch_types=_SC_SCRATCH,
        compiler_params=pltpu.CompilerParams(needs_layout_passes=False),
        interpret=interpret,
    )(_sc_encode_body)


_sc_encode = _make_sc_encode()


BN = 1024


def _mlp_body(encT, posT, dirT, w1t, w2t, hw1t, hw2t, hw3t, rgbT, denT):
    e = encT[...]
    h = jnp.maximum(jnp.dot(w1t[...], e, preferred_element_type=jnp.float32), 0.0)
    out = jnp.dot(w2t[...], h, preferred_element_type=jnp.float32)

    p = posT[...]
    x = (p + 1.0) / 2.0
    x0, x1, x2 = x[0:1, :], x[1:2, :], x[2:3, :]
    sel = (x0 > 0.0) & (x0 < 1.0) & (x1 > 0.0) & (x1 < 1.0) & (x2 > 0.0) & (x2 < 1.0)
    den = jnp.exp(out[0:1, :] - 1.0) * sel.astype(jnp.float32)

    d = dirT[...]
    dx, dy, dz = d[0:1, :], d[1:2, :], d[2:3, :]
    xx, yy, zz = dx * dx, dy * dy, dz * dz
    comps = [
        jnp.full_like(dx, 0.28209479177387814),
        -0.48860251190291987 * dy,
        0.48860251190291987 * dz,
        -0.48860251190291987 * dx,
        1.0925484305920792 * dx * dy,
        -1.0925484305920792 * dy * dz,
        0.94617469575755997 * zz - 0.31539156525251999,
        -1.0925484305920792 * dx * dz,
        0.54627421529603959 * (xx - yy),
        0.59004358992664352 * dy * (3.0 * xx - yy),
        2.8906114426405538 * dx * dy * dz,
        0.45704579946446572 * dy * (4.0 * zz - xx - yy),
        0.3731763325901154 * dz * (2.0 * zz - 3.0 * xx - 3.0 * yy),
        0.45704579946446572 * dx * (4.0 * zz - xx - yy),
        1.4453057213202769 * dz * (xx - yy),
        0.59004358992664352 * dx * (xx - 3.0 * yy),
    ]
    sh = jnp.concatenate(comps, axis=0)
    hh = jnp.concatenate([sh, out[1:16, :], jnp.zeros((1, BN), jnp.float32)], axis=0)
    h1 = jnp.maximum(jnp.dot(hw1t[...], hh, preferred_element_type=jnp.float32), 0.0)
    h2 = jnp.maximum(jnp.dot(hw2t[...], h1, preferred_element_type=jnp.float32), 0.0)
    z = jnp.dot(hw3t[...], h2, preferred_element_type=jnp.float32)
    rgbT[...] = 1.0 / (1.0 + jnp.exp(-z))
    denT[...] = den


def _mlp(encT, posT, dirT, w1t, w2t, hw1t, hw2t, hw3t):
    grid = (N // BN,)
    return pl.pallas_call(
        _mlp_body,
        grid=grid,
        in_specs=[
            pl.BlockSpec((2 * N_LEVELS, BN), lambda i: (0, i)),
            pl.BlockSpec((3, BN), lambda i: (0, i)),
            pl.BlockSpec((3, BN), lambda i: (0, i)),
            pl.BlockSpec((64, 32), lambda i: (0, 0)),
            pl.BlockSpec((16, 64), lambda i: (0, 0)),
            pl.BlockSpec((64, 32), lambda i: (0, 0)),
            pl.BlockSpec((64, 64), lambda i: (0, 0)),
            pl.BlockSpec((3, 64), lambda i: (0, 0)),
        ],
        out_specs=[
            pl.BlockSpec((3, BN), lambda i: (0, i)),
            pl.BlockSpec((1, BN), lambda i: (0, i)),
        ],
        out_shape=[
            jax.ShapeDtypeStruct((3, N), jnp.float32),
            jax.ShapeDtypeStruct((1, N), jnp.float32),
        ],
    )(encT, posT, dirT, w1t, w2t, hw1t, hw2t, hw3t)


def kernel(positions, directions, hash_table, base_W1, base_W2, head_W1, head_W2, head_W3):
    posT = positions.T
    dirT = directions.T
    packed = lax.bitcast_convert_type(
        hash_table.astype(jnp.bfloat16), jnp.int32
    ).reshape(N_LEVELS * T)
    encT = _sc_encode(packed, posT)
    w1t = base_W1.T
    w2t = base_W2.T
    hw1t = jnp.pad(head_W1, ((0, 1), (0, 0))).T
    hw2t = head_W2.T
    hw3t = head_W3.T
    rgbT, denT = _mlp(encT, posT, dirT, w1t, w2t, hw1t, hw2t, hw3t)
    return rgbT.T, denT.T


# speedup vs baseline: 7.9103x; 1.5277x over previous
import functools

import numpy as np
import jax
import jax.numpy as jnp
from jax import lax
from jax.experimental import pallas as pl
from jax.experimental.pallas import tpu as pltpu
from jax.experimental.pallas import tpu_sc as plsc

N = 262144
N_LEVELS = 16
LOG2_T = 19
T = 1 << LOG2_T
MASK = T - 1
SCALE = float(np.exp((np.log(4096.0) - np.log(16.0)) / (N_LEVELS - 1)))
RES = [float(np.floor(16.0 * SCALE ** l)) for l in range(N_LEVELS)]
P1I = int(np.uint32(2654435761).astype(np.int32))
P2I = int(np.uint32(805459861).astype(np.int32))

NW = 32
SPT = N // NW
B = 128
NG = B // 16
NCHUNK = SPT // B


def _sc_encode_body(table_hbm, posT_hbm, out_hbm, xn_v,
                    i0, i1, i2, i3, i4, i5, i6, i7,
                    i8, i9, i10, i11, i12, i13, i14, i15,
                    w_v, words_v, enc_v, sem0, sem1):
    idx_refs = (i0, i1, i2, i3, i4, i5, i6, i7,
                i8, i9, i10, i11, i12, i13, i14, i15)
    sems = (sem0, sem1)
    wid = lax.axis_index("s") * 2 + lax.axis_index("c")
    base0 = wid * SPT

    def make_idx(l, p):
        res = RES[l]
        lbase = l * T

        def idx_body(g, c):
            pid = []
            fr = []
            for d in range(3):
                posd = xn_v[d, pl.ds(g * 16, 16)] * res
                pi = posd.astype(jnp.int32)
                pid.append(pi)
                fr.append(posd - pi.astype(jnp.float32))
            hx = pid[0]
            hy = pid[1] * P1I
            hz = pid[2] * P2I
            hx1 = hx + 1
            hy1 = hy + P1I
            hz1 = hz + P2I
            exy = [hx ^ hy, hx1 ^ hy, hx ^ hy1, hx1 ^ hy1]
            fx1, fy1, fz1 = fr
            fx0 = 1.0 - fx1
            fy0 = 1.0 - fy1
            fz0 = 1.0 - fz1
            wxy = [fx0 * fy0, fx1 * fy0, fx0 * fy1, fx1 * fy1]
            for cc in range(8):
                oz = (cc >> 2) & 1
                k = cc & 3
                h = exy[k] ^ (hz1 if oz else hz)
                idx_refs[p * 8 + cc][pl.ds(g * 16, 16)] = (h & MASK) + lbase
                w_v[pl.ds((p * 8 + cc) * B + g * 16, 16)] = (
                    wxy[k] * (fz1 if oz else fz0)
                )
            return c

        lax.fori_loop(0, NG, idx_body, 0)

    def issue(p):
        for cc in range(8):
            pltpu.async_copy(
                table_hbm.at[idx_refs[p * 8 + cc]],
                words_v.at[pl.ds((p * 8 + cc) * B, B)],
                sems[p],
            )

    def drain(p):
        pltpu.make_async_copy(
            table_hbm.at[pl.ds(0, 8 * B)],
            words_v.at[pl.ds(p * 8 * B, 8 * B)],
            sems[p],
        ).wait()

    def comb(l, p):
        def comb_body(g, c):
            acc0 = jnp.zeros((16,), jnp.float32)
            acc1 = jnp.zeros((16,), jnp.float32)
            for cc in range(8):
                off = (p * 8 + cc) * B + g * 16
                wv = w_v[pl.ds(off, 16)]
                word = words_v[pl.ds(off, 16)]
                bf = plsc.bitcast(word, jnp.bfloat16)
                e0, e1 = plsc.unpack(bf, format=plsc.PackFormat.INTERLEAVED)
                acc0 = acc0 + wv * e0
                acc1 = acc1 + wv * e1
            enc_v[2 * l, pl.ds(g * 16, 16)] = acc0
            enc_v[2 * l + 1, pl.ds(g * 16, 16)] = acc1
            return c

        lax.fori_loop(0, NG, comb_body, 0)

    def chunk_body(ci, carry):
        cbase = base0 + ci * B
        pltpu.sync_copy(posT_hbm.at[:, pl.ds(cbase, B)], xn_v)

        def norm_body(g, c):
            for d in range(3):
                p = xn_v[d, pl.ds(g * 16, 16)]
                xn_v[d, pl.ds(g * 16, 16)] = (p + 1.0) * 0.5
            return c

        lax.fori_loop(0, NG, norm_body, 0)

        make_idx(0, 0)
        issue(0)
        for l in range(1, N_LEVELS):
            p = l & 1
            q = (l - 1) & 1
            make_idx(l, p)
            issue(p)
            drain(q)
            comb(l - 1, q)
        drain(1)
        comb(N_LEVELS - 1, 1)

        pltpu.sync_copy(enc_v, out_hbm.at[:, pl.ds(cbase, B)])
        return carry

    lax.fori_loop(0, NCHUNK, chunk_body, 0)


_SC_SCRATCH = [
    pltpu.VMEM((3, B), jnp.float32),
    *[pltpu.VMEM((B,), jnp.int32) for _ in range(16)],
    pltpu.VMEM((16 * B,), jnp.float32),
    pltpu.VMEM((16 * B,), jnp.int32),
    pltpu.VMEM((2 * N_LEVELS, B), jnp.float32),
    pltpu.SemaphoreType.DMA,
    pltpu.SemaphoreType.DMA,
]


def _make_sc_encode(interpret=False):
    return functools.partial(
        pl.kernel,
        out_type=jax.ShapeDtypeStruct((2 * N_LEVELS, N), jnp.float32),
        mesh=plsc.VectorSubcoreMesh(
            core_axis_name="c", subcore_axis_name="s", num_cores=2, num_subcores=16
        ),
        scratch_types=_SC_SCRATCH,
        compiler_params=pltpu.CompilerParams(needs_layout_passes=False),
        interpret=interpret,
    )(_sc_encode_body)


_sc_encode = _make_sc_encode()


BN = 1024


def _mlp_body(encT, posT, dirT, w1t, w2t, hw1t, hw2t, hw3t, rgbT, denT):
    e = encT[...]
    h = jnp.maximum(jnp.dot(w1t[...], e, preferred_element_type=jnp.float32), 0.0)
    out = jnp.dot(w2t[...], h, preferred_element_type=jnp.float32)

    p = posT[...]
    x = (p + 1.0) / 2.0
    x0, x1, x2 = x[0:1, :], x[1:2, :], x[2:3, :]
    sel = (x0 > 0.0) & (x0 < 1.0) & (x1 > 0.0) & (x1 < 1.0) & (x2 > 0.0) & (x2 < 1.0)
    den = jnp.exp(out[0:1, :] - 1.0) * sel.astype(jnp.float32)

    d = dirT[...]
    dx, dy, dz = d[0:1, :], d[1:2, :], d[2:3, :]
    xx, yy, zz = dx * dx, dy * dy, dz * dz
    comps = [
        jnp.full_like(dx, 0.28209479177387814),
        -0.48860251190291987 * dy,
        0.48860251190291987 * dz,
        -0.48860251190291987 * dx,
        1.0925484305920792 * dx * dy,
        -1.0925484305920792 * dy * dz,
        0.94617469575755997 * zz - 0.31539156525251999,
        -1.0925484305920792 * dx * dz,
        0.54627421529603959 * (xx - yy),
        0.59004358992664352 * dy * (3.0 * xx - yy),
        2.8906114426405538 * dx * dy * dz,
        0.45704579946446572 * dy * (4.0 * zz - xx - yy),
        0.3731763325901154 * dz * (2.0 * zz - 3.0 * xx - 3.0 * yy),
        0.45704579946446572 * dx * (4.0 * zz - xx - yy),
        1.4453057213202769 * dz * (xx - yy),
        0.59004358992664352 * dx * (xx - 3.0 * yy),
    ]
    sh = jnp.concatenate(comps, axis=0)
    hh = jnp.concatenate([sh, out[1:16, :], jnp.zeros((1, BN), jnp.float32)], axis=0)
    h1 = jnp.maximum(jnp.dot(hw1t[...], hh, preferred_element_type=jnp.float32), 0.0)
    h2 = jnp.maximum(jnp.dot(hw2t[...], h1, preferred_element_type=jnp.float32), 0.0)
    z = jnp.dot(hw3t[...], h2, preferred_element_type=jnp.float32)
    rgbT[...] = 1.0 / (1.0 + jnp.exp(-z))
    denT[...] = den


def _mlp(encT, posT, dirT, w1t, w2t, hw1t, hw2t, hw3t):
    grid = (N // BN,)
    return pl.pallas_call(
        _mlp_body,
        grid=grid,
        in_specs=[
            pl.BlockSpec((2 * N_LEVELS, BN), lambda i: (0, i)),
            pl.BlockSpec((3, BN), lambda i: (0, i)),
            pl.BlockSpec((3, BN), lambda i: (0, i)),
            pl.BlockSpec((64, 32), lambda i: (0, 0)),
            pl.BlockSpec((16, 64), lambda i: (0, 0)),
            pl.BlockSpec((64, 32), lambda i: (0, 0)),
            pl.BlockSpec((64, 64), lambda i: (0, 0)),
            pl.BlockSpec((3, 64), lambda i: (0, 0)),
        ],
        out_specs=[
            pl.BlockSpec((3, BN), lambda i: (0, i)),
            pl.BlockSpec((1, BN), lambda i: (0, i)),
        ],
        out_shape=[
            jax.ShapeDtypeStruct((3, N), jnp.float32),
            jax.ShapeDtypeStruct((1, N), jnp.float32),
        ],
    )(encT, posT, dirT, w1t, w2t, hw1t, hw2t, hw3t)


def kernel(positions, directions, hash_table, base_W1, base_W2, head_W1, head_W2, head_W3):
    posT = positions.T
    dirT = directions.T
    packed = lax.bitcast_convert_type(
        hash_table.astype(jnp.bfloat16), jnp.int32
    ).reshape(N_LEVELS * T)
    encT = _sc_encode(packed, posT)
    w1t = base_W1.T
    w2t = base_W2.T
    hw1t = jnp.pad(head_W1, ((0, 1), (0, 0))).T
    hw2t = head_W2.T
    hw3t = head_W3.T
    rgbT, denT = _mlp(encT, posT, dirT, w1t, w2t, hw1t, hw2t, hw3t)
    return rgbT.T, denT.T


# speedup vs baseline: 8.6893x; 1.0985x over previous
import functools

import numpy as np
import jax
import jax.numpy as jnp
from jax import lax
from jax.experimental import pallas as pl
from jax.experimental.pallas import tpu as pltpu
from jax.experimental.pallas import tpu_sc as plsc

N = 262144
N_LEVELS = 16
LOG2_T = 19
T = 1 << LOG2_T
MASK = T - 1
SCALE = float(np.exp((np.log(4096.0) - np.log(16.0)) / (N_LEVELS - 1)))
RES = [float(np.floor(16.0 * SCALE ** l)) for l in range(N_LEVELS)]
P1I = int(np.uint32(2654435761).astype(np.int32))
P2I = int(np.uint32(805459861).astype(np.int32))

NSH = 3
NW = 32
SPT = N // NW
B = 128
NG = B // 16
NCHUNK = SPT // B


def _sc_encode_body(table_hbm, posT_hbm, out_hbm, xn_v,
                    i0, i1, i2, i3, i4, i5, i6, i7,
                    i8, i9, i10, i11, i12, i13, i14, i15,
                    w_v, words_v, enc_v, shared_v, sem0, sem1):
    idx_refs = (i0, i1, i2, i3, i4, i5, i6, i7,
                i8, i9, i10, i11, i12, i13, i14, i15)
    sems = (sem0, sem1)
    sid = lax.axis_index("s")
    wid = sid * 2 + lax.axis_index("c")
    base0 = wid * SPT

    @pl.when(sid == 0)
    def _stage():
        pltpu.sync_copy(table_hbm.at[pl.ds(0, NSH * T)], shared_v)

    plsc.subcore_barrier()

    def make_idx(l, p):
        res = RES[l]
        lbase = l * T

        def idx_body(g, c):
            pid = []
            fr = []
            for d in range(3):
                posd = xn_v[d, pl.ds(g * 16, 16)] * res
                pi = posd.astype(jnp.int32)
                pid.append(pi)
                fr.append(posd - pi.astype(jnp.float32))
            hx = pid[0]
            hy = pid[1] * P1I
            hz = pid[2] * P2I
            hx1 = hx + 1
            hy1 = hy + P1I
            hz1 = hz + P2I
            exy = [hx ^ hy, hx1 ^ hy, hx ^ hy1, hx1 ^ hy1]
            fx1, fy1, fz1 = fr
            fx0 = 1.0 - fx1
            fy0 = 1.0 - fy1
            fz0 = 1.0 - fz1
            wxy = [fx0 * fy0, fx1 * fy0, fx0 * fy1, fx1 * fy1]
            for cc in range(8):
                oz = (cc >> 2) & 1
                k = cc & 3
                h = exy[k] ^ (hz1 if oz else hz)
                idx_refs[p * 8 + cc][pl.ds(g * 16, 16)] = (h & MASK) + lbase
                w_v[pl.ds((p * 8 + cc) * B + g * 16, 16)] = (
                    wxy[k] * (fz1 if oz else fz0)
                )
            return c

        lax.fori_loop(0, NG, idx_body, 0)

    def issue(l, p):
        src = shared_v if l < NSH else table_hbm
        for cc in range(8):
            pltpu.async_copy(
                src.at[idx_refs[p * 8 + cc]],
                words_v.at[pl.ds((p * 8 + cc) * B, B)],
                sems[p],
            )

    def drain(p):
        pltpu.make_async_copy(
            table_hbm.at[pl.ds(0, 8 * B)],
            words_v.at[pl.ds(p * 8 * B, 8 * B)],
            sems[p],
        ).wait()

    def comb(l, p):
        def comb_body(g, c):
            acc0 = jnp.zeros((16,), jnp.float32)
            acc1 = jnp.zeros((16,), jnp.float32)
            for cc in range(8):
                off = (p * 8 + cc) * B + g * 16
                wv = w_v[pl.ds(off, 16)]
                word = words_v[pl.ds(off, 16)]
                bf = plsc.bitcast(word, jnp.bfloat16)
                e0, e1 = plsc.unpack(bf, format=plsc.PackFormat.INTERLEAVED)
                acc0 = acc0 + wv * e0
                acc1 = acc1 + wv * e1
            enc_v[2 * l, pl.ds(g * 16, 16)] = acc0
            enc_v[2 * l + 1, pl.ds(g * 16, 16)] = acc1
            return c

        lax.fori_loop(0, NG, comb_body, 0)

    def chunk_body(ci, carry):
        cbase = base0 + ci * B
        pltpu.sync_copy(posT_hbm.at[:, pl.ds(cbase, B)], xn_v)

        def norm_body(g, c):
            for d in range(3):
                p = xn_v[d, pl.ds(g * 16, 16)]
                xn_v[d, pl.ds(g * 16, 16)] = (p + 1.0) * 0.5
            return c

        lax.fori_loop(0, NG, norm_body, 0)

        make_idx(0, 0)
        issue(0, 0)
        for l in range(1, N_LEVELS):
            p = l & 1
            q = (l - 1) & 1
            make_idx(l, p)
            issue(l, p)
            drain(q)
            comb(l - 1, q)
        drain(1)
        comb(N_LEVELS - 1, 1)

        pltpu.sync_copy(enc_v, out_hbm.at[:, pl.ds(cbase, B)])
        return carry

    lax.fori_loop(0, NCHUNK, chunk_body, 0)


_SC_SCRATCH = [
    pltpu.VMEM((3, B), jnp.float32),
    *[pltpu.VMEM((B,), jnp.int32) for _ in range(16)],
    pltpu.VMEM((16 * B,), jnp.float32),
    pltpu.VMEM((16 * B,), jnp.int32),
    pltpu.VMEM((2 * N_LEVELS, B), jnp.float32),
    pltpu.VMEM_SHARED((NSH * T,), jnp.int32),
    pltpu.SemaphoreType.DMA,
    pltpu.SemaphoreType.DMA,
]


def _make_sc_encode(interpret=False):
    return functools.partial(
        pl.kernel,
        out_type=jax.ShapeDtypeStruct((2 * N_LEVELS, N), jnp.float32),
        mesh=plsc.VectorSubcoreMesh(
            core_axis_name="c", subcore_axis_name="s", num_cores=2, num_subcores=16
        ),
        scratch_types=_SC_SCRATCH,
        compiler_params=pltpu.CompilerParams(needs_layout_passes=False),
        interpret=interpret,
    )(_sc_encode_body)


_sc_encode = _make_sc_encode()


BN = 1024


def _mlp_body(encT, posT, dirT, w1t, w2t, hw1t, hw2t, hw3t, rgbT, denT):
    e = encT[...]
    h = jnp.maximum(jnp.dot(w1t[...], e, preferred_element_type=jnp.float32), 0.0)
    out = jnp.dot(w2t[...], h, preferred_element_type=jnp.float32)

    p = posT[...]
    x = (p + 1.0) / 2.0
    x0, x1, x2 = x[0:1, :], x[1:2, :], x[2:3, :]
    sel = (x0 > 0.0) & (x0 < 1.0) & (x1 > 0.0) & (x1 < 1.0) & (x2 > 0.0) & (x2 < 1.0)
    den = jnp.exp(out[0:1, :] - 1.0) * sel.astype(jnp.float32)

    d = dirT[...]
    dx, dy, dz = d[0:1, :], d[1:2, :], d[2:3, :]
    xx, yy, zz = dx * dx, dy * dy, dz * dz
    comps = [
        jnp.full_like(dx, 0.28209479177387814),
        -0.48860251190291987 * dy,
        0.48860251190291987 * dz,
        -0.48860251190291987 * dx,
        1.0925484305920792 * dx * dy,
        -1.0925484305920792 * dy * dz,
        0.94617469575755997 * zz - 0.31539156525251999,
        -1.0925484305920792 * dx * dz,
        0.54627421529603959 * (xx - yy),
        0.59004358992664352 * dy * (3.0 * xx - yy),
        2.8906114426405538 * dx * dy * dz,
        0.45704579946446572 * dy * (4.0 * zz - xx - yy),
        0.3731763325901154 * dz * (2.0 * zz - 3.0 * xx - 3.0 * yy),
        0.45704579946446572 * dx * (4.0 * zz - xx - yy),
        1.4453057213202769 * dz * (xx - yy),
        0.59004358992664352 * dx * (xx - 3.0 * yy),
    ]
    sh = jnp.concatenate(comps, axis=0)
    hh = jnp.concatenate([sh, out[1:16, :], jnp.zeros((1, BN), jnp.float32)], axis=0)
    h1 = jnp.maximum(jnp.dot(hw1t[...], hh, preferred_element_type=jnp.float32), 0.0)
    h2 = jnp.maximum(jnp.dot(hw2t[...], h1, preferred_element_type=jnp.float32), 0.0)
    z = jnp.dot(hw3t[...], h2, preferred_element_type=jnp.float32)
    rgbT[...] = 1.0 / (1.0 + jnp.exp(-z))
    denT[...] = den


def _mlp(encT, posT, dirT, w1t, w2t, hw1t, hw2t, hw3t):
    grid = (N // BN,)
    return pl.pallas_call(
        _mlp_body,
        grid=grid,
        in_specs=[
            pl.BlockSpec((2 * N_LEVELS, BN), lambda i: (0, i)),
            pl.BlockSpec((3, BN), lambda i: (0, i)),
            pl.BlockSpec((3, BN), lambda i: (0, i)),
            pl.BlockSpec((64, 32), lambda i: (0, 0)),
            pl.BlockSpec((16, 64), lambda i: (0, 0)),
            pl.BlockSpec((64, 32), lambda i: (0, 0)),
            pl.BlockSpec((64, 64), lambda i: (0, 0)),
            pl.BlockSpec((3, 64), lambda i: (0, 0)),
        ],
        out_specs=[
            pl.BlockSpec((3, BN), lambda i: (0, i)),
            pl.BlockSpec((1, BN), lambda i: (0, i)),
        ],
        out_shape=[
            jax.ShapeDtypeStruct((3, N), jnp.float32),
            jax.ShapeDtypeStruct((1, N), jnp.float32),
        ],
    )(encT, posT, dirT, w1t, w2t, hw1t, hw2t, hw3t)


def kernel(positions, directions, hash_table, base_W1, base_W2, head_W1, head_W2, head_W3):
    posT = positions.T
    dirT = directions.T
    packed = lax.bitcast_convert_type(
        hash_table.astype(jnp.bfloat16), jnp.int32
    ).reshape(N_LEVELS * T)
    encT = _sc_encode(packed, posT)
    w1t = base_W1.T
    w2t = base_W2.T
    hw1t = jnp.pad(head_W1, ((0, 1), (0, 0))).T
    hw2t = head_W2.T
    hw3t = head_W3.T
    rgbT, denT = _mlp(encT, posT, dirT, w1t, w2t, hw1t, hw2t, hw3t)
    return rgbT.T, denT.T


# speedup vs baseline: 8.9219x; 1.0268x over previous
import functools

import numpy as np
import jax
import jax.numpy as jnp
from jax import lax
from jax.experimental import pallas as pl
from jax.experimental.pallas import tpu as pltpu
from jax.experimental.pallas import tpu_sc as plsc

N = 262144
N_LEVELS = 16
LOG2_T = 19
T = 1 << LOG2_T
MASK = T - 1
SCALE = float(np.exp((np.log(4096.0) - np.log(16.0)) / (N_LEVELS - 1)))
RES = [float(np.floor(16.0 * SCALE ** l)) for l in range(N_LEVELS)]
P1I = int(np.uint32(2654435761).astype(np.int32))
P2I = int(np.uint32(805459861).astype(np.int32))

NSH = 3
NW = 32
SPT = N // NW
B = 128
NG = B // 16
NCHUNK = SPT // B


def _sc_encode_body(table_hbm, posT_hbm, out_hbm, xn_v,
                    i0, i1, i2, i3, i4, i5, i6, i7,
                    i8, i9, i10, i11, i12, i13, i14, i15,
                    w_v, words_v, enc_v, shared_v, sem0, sem1, sem2):
    idx_refs = (i0, i1, i2, i3, i4, i5, i6, i7,
                i8, i9, i10, i11, i12, i13, i14, i15)
    sems = (sem0, sem1)
    sid = lax.axis_index("s")
    wid = sid * 2 + lax.axis_index("c")
    base0 = wid * SPT

    @pl.when(sid == 0)
    def _stage():
        pltpu.sync_copy(table_hbm.at[pl.ds(0, NSH * T)], shared_v)

    plsc.subcore_barrier()

    def make_idx(l, p):
        res = RES[l]
        lbase = l * T

        def idx_body(g, c):
            pid = []
            fr = []
            for d in range(3):
                posd = xn_v[d, pl.ds(g * 16, 16)] * res
                pi = posd.astype(jnp.int32)
                pid.append(pi)
                fr.append(posd - pi.astype(jnp.float32))
            hx = pid[0]
            hy = pid[1] * P1I
            hz = pid[2] * P2I
            hx1 = hx + 1
            hy1 = hy + P1I
            hz1 = hz + P2I
            exy = [hx ^ hy, hx1 ^ hy, hx ^ hy1, hx1 ^ hy1]
            fx1, fy1, fz1 = fr
            fx0 = 1.0 - fx1
            fy0 = 1.0 - fy1
            fz0 = 1.0 - fz1
            wxy = [fx0 * fy0, fx1 * fy0, fx0 * fy1, fx1 * fy1]
            for cc in range(8):
                oz = (cc >> 2) & 1
                k = cc & 3
                h = exy[k] ^ (hz1 if oz else hz)
                idx_refs[p * 8 + cc][pl.ds(g * 16, 16)] = (h & MASK) + lbase
                w_v[pl.ds((p * 8 + cc) * B + g * 16, 16)] = (
                    wxy[k] * (fz1 if oz else fz0)
                )
            return c

        lax.fori_loop(0, NG, idx_body, 0)

    def issue(l, p):
        src = shared_v if l < NSH else table_hbm
        for cc in range(8):
            pltpu.async_copy(
                src.at[idx_refs[p * 8 + cc]],
                words_v.at[pl.ds((p * 8 + cc) * B, B)],
                sems[p],
            )

    def drain(p):
        pltpu.make_async_copy(
            table_hbm.at[pl.ds(0, 8 * B)],
            words_v.at[pl.ds(p * 8 * B, 8 * B)],
            sems[p],
        ).wait()

    def comb(l, p, eoff):
        def comb_body(g, c):
            acc0 = jnp.zeros((16,), jnp.float32)
            acc1 = jnp.zeros((16,), jnp.float32)
            for cc in range(8):
                off = (p * 8 + cc) * B + g * 16
                wv = w_v[pl.ds(off, 16)]
                word = words_v[pl.ds(off, 16)]
                bf = plsc.bitcast(word, jnp.bfloat16)
                e0, e1 = plsc.unpack(bf, format=plsc.PackFormat.INTERLEAVED)
                acc0 = acc0 + wv * e0
                acc1 = acc1 + wv * e1
            enc_v[eoff + 2 * l, pl.ds(g * 16, 16)] = acc0
            enc_v[eoff + 2 * l + 1, pl.ds(g * 16, 16)] = acc1
            return c

        lax.fori_loop(0, NG, comb_body, 0)

    NR = 2 * N_LEVELS

    def chunk_body(ci, carry):
        cbase = base0 + ci * B
        eoff = (ci & 1) * NR
        peoff = NR - eoff
        pltpu.sync_copy(posT_hbm.at[:, pl.ds(cbase, B)], xn_v)

        def norm_body(g, c):
            for d in range(3):
                p = xn_v[d, pl.ds(g * 16, 16)]
                xn_v[d, pl.ds(g * 16, 16)] = (p + 1.0) * 0.5
            return c

        lax.fori_loop(0, NG, norm_body, 0)

        make_idx(0, 0)
        issue(0, 0)

        @pl.when(ci > 1)
        def _wait_prev_out():
            pltpu.make_async_copy(
                enc_v.at[pl.ds(0, NR)],
                out_hbm.at[:, pl.ds(cbase, B)],
                sem2,
            ).wait()

        @pl.when(ci > 0)
        def _finish_prev():
            drain(1)
            comb(N_LEVELS - 1, 1, peoff)
            pltpu.async_copy(
                enc_v.at[pl.ds(peoff, NR)],
                out_hbm.at[:, pl.ds(cbase - B, B)],
                sem2,
            )

        for l in range(1, N_LEVELS):
            p = l & 1
            q = (l - 1) & 1
            make_idx(l, p)
            issue(l, p)
            drain(q)
            comb(l - 1, q, eoff)
        return carry

    lax.fori_loop(0, NCHUNK, chunk_body, 0)

    last = base0 + (NCHUNK - 1) * B
    leoff = ((NCHUNK - 1) & 1) * NR
    pltpu.make_async_copy(
        enc_v.at[pl.ds(0, NR)],
        out_hbm.at[:, pl.ds(last, B)],
        sem2,
    ).wait()
    drain(1)
    comb(N_LEVELS - 1, 1, leoff)
    pltpu.sync_copy(
        enc_v.at[pl.ds(leoff, NR)], out_hbm.at[:, pl.ds(last, B)]
    )


_SC_SCRATCH = [
    pltpu.VMEM((3, B), jnp.float32),
    *[pltpu.VMEM((B,), jnp.int32) for _ in range(16)],
    pltpu.VMEM((16 * B,), jnp.float32),
    pltpu.VMEM((16 * B,), jnp.int32),
    pltpu.VMEM((4 * N_LEVELS, B), jnp.float32),
    pltpu.VMEM_SHARED((NSH * T,), jnp.int32),
    pltpu.SemaphoreType.DMA,
    pltpu.SemaphoreType.DMA,
    pltpu.SemaphoreType.DMA,
]


def _make_sc_encode(interpret=False):
    return functools.partial(
        pl.kernel,
        out_type=jax.ShapeDtypeStruct((2 * N_LEVELS, N), jnp.float32),
        mesh=plsc.VectorSubcoreMesh(
            core_axis_name="c", subcore_axis_name="s", num_cores=2, num_subcores=16
        ),
        scratch_types=_SC_SCRATCH,
        compiler_params=pltpu.CompilerParams(needs_layout_passes=False),
        interpret=interpret,
    )(_sc_encode_body)


_sc_encode = _make_sc_encode()


BN = 1024


def _mlp_body(encT, posT, dirT, w1t, w2t, hw1t, hw2t, hw3t, rgbT, denT):
    e = encT[...]
    h = jnp.maximum(jnp.dot(w1t[...], e, preferred_element_type=jnp.float32), 0.0)
    out = jnp.dot(w2t[...], h, preferred_element_type=jnp.float32)

    p = posT[...]
    x = (p + 1.0) / 2.0
    x0, x1, x2 = x[0:1, :], x[1:2, :], x[2:3, :]
    sel = (x0 > 0.0) & (x0 < 1.0) & (x1 > 0.0) & (x1 < 1.0) & (x2 > 0.0) & (x2 < 1.0)
    den = jnp.exp(out[0:1, :] - 1.0) * sel.astype(jnp.float32)

    d = dirT[...]
    dx, dy, dz = d[0:1, :], d[1:2, :], d[2:3, :]
    xx, yy, zz = dx * dx, dy * dy, dz * dz
    comps = [
        jnp.full_like(dx, 0.28209479177387814),
        -0.48860251190291987 * dy,
        0.48860251190291987 * dz,
        -0.48860251190291987 * dx,
        1.0925484305920792 * dx * dy,
        -1.0925484305920792 * dy * dz,
        0.94617469575755997 * zz - 0.31539156525251999,
        -1.0925484305920792 * dx * dz,
        0.54627421529603959 * (xx - yy),
        0.59004358992664352 * dy * (3.0 * xx - yy),
        2.8906114426405538 * dx * dy * dz,
        0.45704579946446572 * dy * (4.0 * zz - xx - yy),
        0.3731763325901154 * dz * (2.0 * zz - 3.0 * xx - 3.0 * yy),
        0.45704579946446572 * dx * (4.0 * zz - xx - yy),
        1.4453057213202769 * dz * (xx - yy),
        0.59004358992664352 * dx * (xx - 3.0 * yy),
    ]
    sh = jnp.concatenate(comps, axis=0)
    hh = jnp.concatenate([sh, out[1:16, :], jnp.zeros((1, BN), jnp.float32)], axis=0)
    h1 = jnp.maximum(jnp.dot(hw1t[...], hh, preferred_element_type=jnp.float32), 0.0)
    h2 = jnp.maximum(jnp.dot(hw2t[...], h1, preferred_element_type=jnp.float32), 0.0)
    z = jnp.dot(hw3t[...], h2, preferred_element_type=jnp.float32)
    rgbT[...] = 1.0 / (1.0 + jnp.exp(-z))
    denT[...] = den


def _mlp(encT, posT, dirT, w1t, w2t, hw1t, hw2t, hw3t):
    grid = (N // BN,)
    return pl.pallas_call(
        _mlp_body,
        grid=grid,
        in_specs=[
            pl.BlockSpec((2 * N_LEVELS, BN), lambda i: (0, i)),
            pl.BlockSpec((3, BN), lambda i: (0, i)),
            pl.BlockSpec((3, BN), lambda i: (0, i)),
            pl.BlockSpec((64, 32), lambda i: (0, 0)),
            pl.BlockSpec((16, 64), lambda i: (0, 0)),
            pl.BlockSpec((64, 32), lambda i: (0, 0)),
            pl.BlockSpec((64, 64), lambda i: (0, 0)),
            pl.BlockSpec((3, 64), lambda i: (0, 0)),
        ],
        out_specs=[
            pl.BlockSpec((3, BN), lambda i: (0, i)),
            pl.BlockSpec((1, BN), lambda i: (0, i)),
        ],
        out_shape=[
            jax.ShapeDtypeStruct((3, N), jnp.float32),
            jax.ShapeDtypeStruct((1, N), jnp.float32),
        ],
    )(encT, posT, dirT, w1t, w2t, hw1t, hw2t, hw3t)


def kernel(positions, directions, hash_table, base_W1, base_W2, head_W1, head_W2, head_W3):
    posT = positions.T
    dirT = directions.T
    packed = lax.bitcast_convert_type(
        hash_table.astype(jnp.bfloat16), jnp.int32
    ).reshape(N_LEVELS * T)
    encT = _sc_encode(packed, posT)
    w1t = base_W1.T
    w2t = base_W2.T
    hw1t = jnp.pad(head_W1, ((0, 1), (0, 0))).T
    hw2t = head_W2.T
    hw3t = head_W3.T
    rgbT, denT = _mlp(encT, posT, dirT, w1t, w2t, hw1t, hw2t, hw3t)
    return rgbT.T, denT.T
